# Initial kernel scaffold; baseline (speedup 1.0000x reference)
import jax
import jax.numpy as jnp
from jax.experimental import pallas as pl


def kernel(X_train, X_test):
    raise NotImplementedError("write your pallas kernel here")


# speedup vs baseline: 6.7662x; 6.7662x over previous
import functools

import jax
import jax.numpy as jnp
from jax import lax
from jax.experimental import pallas as pl
from jax.experimental.pallas import tpu as pltpu
from jax.experimental.pallas import tpu_sc as plsc

Q = 4096
N_TRAIN = 100000
D = 128
K = 16

KB = 4096
NKB = 25
NPAD = KB * NKB
CHUNK = 256
NCHUNK = KB // CHUNK
NSEG = NKB * CHUNK
QB1 = 256
QB2 = 256
PAD_VAL = 1e4

_BIG = 3e38


def _phase1_body(xt_ref, tr_ref, dist_ref, m_ref):
    xt = xt_ref[...]
    tr = tr_ref[...]
    x2 = jnp.sum(xt * xt, axis=1, keepdims=True)
    y2 = jnp.sum(tr * tr, axis=1)[None, :]
    prod = lax.dot_general(xt, tr, (((1,), (1,)), ((), ())),
                           preferred_element_type=jnp.float32)
    d2 = x2 + y2 - 2.0 * prod
    dist = jnp.sqrt(jnp.maximum(d2, 0.0))
    dist_ref[...] = dist
    m = dist[:, 0:CHUNK]
    for i in range(1, NCHUNK):
        m = jnp.minimum(m, dist[:, i * CHUNK:(i + 1) * CHUNK])
    m_ref[...] = m


def _phase2_body(m_ref, seg_ref):
    m = m_ref[...]
    iota = lax.broadcasted_iota(jnp.int32, (QB2, NSEG), 1)
    cols = []
    for _ in range(K):
        v = jnp.min(m, axis=1, keepdims=True)
        idx = jnp.min(jnp.where(m == v, iota, NSEG), axis=1, keepdims=True)
        cols.append(idx)
        m = jnp.where(iota == idx, _BIG, m)
    seg_ref[...] = jnp.concatenate(cols, axis=1)


def _sc_merge16(bv, bc, v, c):
    rv = lax.rev(v, (0,))
    rc = lax.rev(c, (0,))
    keep = bv <= rv
    nv = jnp.where(keep, bv, rv)
    nc = jnp.where(keep, bc, rc)
    return plsc.sort_key_val(nv, nc)


def _phase3_sc(dist_flat, seg_ids):
    info = plsc.get_sparse_core_info()
    nw = info.num_cores * info.num_subcores
    qpw = Q // nw
    mesh = plsc.VectorSubcoreMesh(core_axis_name="c", subcore_axis_name="s")

    @functools.partial(
        pl.kernel, mesh=mesh,
        out_type=jax.ShapeDtypeStruct((Q, K), jnp.int32),
        compiler_params=pltpu.CompilerParams(needs_layout_passes=False),
        scratch_types=[
            pltpu.VMEM((K,), jnp.int32),
            pltpu.VMEM((K * K,), jnp.int32),
            pltpu.VMEM((K * K,), jnp.int32),
            pltpu.VMEM((K * K,), jnp.float32),
            pltpu.VMEM((K,), jnp.int32),
            pltpu.SemaphoreType.DMA,
        ],
    )
    def sc_kernel(dist_hbm, seg_hbm, out_hbm, segv, addrv, colv, valv,
                  outv, sem):
        wid = lax.axis_index("s") * info.num_cores + lax.axis_index("c")
        q0 = wid * qpw

        def body(j, carry):
            q = q0 + j
            pltpu.sync_copy(seg_hbm.at[q], segv)
            seg = segv[...]
            col_base = ((seg >> 8) * KB) + (seg & (CHUNK - 1))
            qbase = q * NPAD
            for i in range(NCHUNK):
                c = col_base + i * CHUNK
                colv[pl.ds(i * 16, 16)] = c
                addrv[pl.ds(i * 16, 16)] = c + qbase
            pltpu.async_copy(dist_hbm.at[addrv], valv, sem).wait()
            bv, bc = plsc.sort_key_val(valv[pl.ds(0, 16)],
                                       colv[pl.ds(0, 16)])
            for t in range(1, K):
                v, c = plsc.sort_key_val(valv[pl.ds(16 * t, 16)],
                                         colv[pl.ds(16 * t, 16)])
                bv, bc = _sc_merge16(bv, bc, v, c)
            outv[...] = bc
            pltpu.sync_copy(outv, out_hbm.at[q])
            return carry

        lax.fori_loop(0, qpw, body, 0)

    return sc_kernel(dist_flat, seg_ids)


def kernel(X_train, X_test):
    pad = jnp.full((NPAD - N_TRAIN, D), PAD_VAL, dtype=jnp.float32)
    tr = jnp.concatenate([X_train, pad], axis=0)

    dist, m = pl.pallas_call(
        _phase1_body,
        grid=(NKB, Q // QB1),
        in_specs=[
            pl.BlockSpec((QB1, D), lambda k, q: (q, 0)),
            pl.BlockSpec((KB, D), lambda k, q: (k, 0)),
        ],
        out_specs=[
            pl.BlockSpec((QB1, KB), lambda k, q: (q, k)),
            pl.BlockSpec((QB1, CHUNK), lambda k, q: (q, k)),
        ],
        out_shape=[
            jax.ShapeDtypeStruct((Q, NPAD), jnp.float32),
            jax.ShapeDtypeStruct((Q, NSEG), jnp.float32),
        ],
    )(X_test, tr)

    seg_ids = pl.pallas_call(
        _phase2_body,
        grid=(Q // QB2,),
        in_specs=[pl.BlockSpec((QB2, NSEG), lambda q: (q, 0))],
        out_specs=pl.BlockSpec((QB2, K), lambda q: (q, 0)),
        out_shape=jax.ShapeDtypeStruct((Q, K), jnp.int32),
    )(m)

    return _phase3_sc(dist.reshape(Q * NPAD), seg_ids)


# speedup vs baseline: 6.8535x; 1.0129x over previous
import functools

import jax
import jax.numpy as jnp
from jax import lax
from jax.experimental import pallas as pl
from jax.experimental.pallas import tpu as pltpu
from jax.experimental.pallas import tpu_sc as plsc

Q = 4096
N_TRAIN = 100000
D = 128
K = 16

KB = 4096
NKB = 25
NPAD = KB * NKB
CHUNK = 256
NCHUNK = KB // CHUNK
NSEG = NKB * CHUNK
QB1 = 256
QB2 = 256
PAD_VAL = 1e4

_BIG = 3e38


def _phase1_body(xt_ref, tr_ref, dist_ref, m_ref):
    xt = xt_ref[...]
    tr = tr_ref[...]
    x2 = jnp.sum(xt * xt, axis=1, keepdims=True)
    y2 = jnp.sum(tr * tr, axis=1)[None, :]
    prod = lax.dot_general(xt, tr, (((1,), (1,)), ((), ())),
                           preferred_element_type=jnp.float32)
    d2 = x2 + y2 - 2.0 * prod
    dist = jnp.sqrt(jnp.maximum(d2, 0.0))
    dist_ref[0, 0] = dist
    m = dist[:, 0:CHUNK]
    for i in range(1, NCHUNK):
        m = jnp.minimum(m, dist[:, i * CHUNK:(i + 1) * CHUNK])
    m_ref[...] = m


def _phase2_body(m_ref, seg_ref):
    m = m_ref[...]
    iota = lax.broadcasted_iota(jnp.int32, (QB2, NSEG), 1)
    cols = []
    for _ in range(K):
        v = jnp.min(m, axis=1, keepdims=True)
        idx = jnp.min(jnp.where(m == v, iota, NSEG), axis=1, keepdims=True)
        cols.append(idx)
        m = jnp.where(iota == idx, _BIG, m)
    seg_ref[...] = jnp.concatenate(cols, axis=1)


def _sc_merge16(bv, bc, v, c):
    rv = lax.rev(v, (0,))
    rc = lax.rev(c, (0,))
    keep = bv <= rv
    nv = jnp.where(keep, bv, rv)
    nc = jnp.where(keep, bc, rc)
    return plsc.sort_key_val(nv, nc)


def _phase3_sc(dist_flat, seg_ids):
    info = plsc.get_sparse_core_info()
    nw = info.num_cores * info.num_subcores
    qpw = Q // nw
    mesh = plsc.VectorSubcoreMesh(core_axis_name="c", subcore_axis_name="s")

    @functools.partial(
        pl.kernel, mesh=mesh,
        out_type=jax.ShapeDtypeStruct((Q, K), jnp.int32),
        compiler_params=pltpu.CompilerParams(needs_layout_passes=False),
        scratch_types=[
            pltpu.VMEM((K,), jnp.int32),
            pltpu.VMEM((K * K,), jnp.int32),
            pltpu.VMEM((K * K,), jnp.int32),
            pltpu.VMEM((K * K,), jnp.float32),
            pltpu.VMEM((K,), jnp.int32),
            pltpu.SemaphoreType.DMA,
        ],
    )
    def sc_kernel(dist_hbm, seg_hbm, out_hbm, segv, addrv, colv, valv,
                  outv, sem):
        wid = lax.axis_index("s") * info.num_cores + lax.axis_index("c")
        q0 = wid * qpw

        def body(j, carry):
            q = q0 + j
            pltpu.sync_copy(seg_hbm.at[q], segv)
            seg = segv[...]
            col_base = ((seg >> 8) * KB) + (seg & (CHUNK - 1))
            qoff = ((q >> 8) << 20) + ((q & (QB1 - 1)) << 12)
            addr_base = ((seg >> 8) << 24) + (seg & (CHUNK - 1)) + qoff
            for i in range(NCHUNK):
                colv[pl.ds(i * 16, 16)] = col_base + i * CHUNK
                addrv[pl.ds(i * 16, 16)] = addr_base + i * CHUNK
            pltpu.async_copy(dist_hbm.at[addrv], valv, sem).wait()
            bv, bc = plsc.sort_key_val(valv[pl.ds(0, 16)],
                                       colv[pl.ds(0, 16)])
            for t in range(1, K):
                v, c = plsc.sort_key_val(valv[pl.ds(16 * t, 16)],
                                         colv[pl.ds(16 * t, 16)])
                bv, bc = _sc_merge16(bv, bc, v, c)
            outv[...] = bc
            pltpu.sync_copy(outv, out_hbm.at[q])
            return carry

        lax.fori_loop(0, qpw, body, 0)

    return sc_kernel(dist_flat, seg_ids)


def kernel(X_train, X_test):
    pad = jnp.full((NPAD - N_TRAIN, D), PAD_VAL, dtype=jnp.float32)
    tr = jnp.concatenate([X_train, pad], axis=0)

    dist, m = pl.pallas_call(
        _phase1_body,
        grid=(NKB, Q // QB1),
        in_specs=[
            pl.BlockSpec((QB1, D), lambda k, q: (q, 0)),
            pl.BlockSpec((KB, D), lambda k, q: (k, 0)),
        ],
        out_specs=[
            pl.BlockSpec((1, 1, QB1, KB), lambda k, q: (k, q, 0, 0)),
            pl.BlockSpec((QB1, CHUNK), lambda k, q: (q, k)),
        ],
        out_shape=[
            jax.ShapeDtypeStruct((NKB, Q // QB1, QB1, KB), jnp.float32),
            jax.ShapeDtypeStruct((Q, NSEG), jnp.float32),
        ],
    )(X_test, tr)

    seg_ids = pl.pallas_call(
        _phase2_body,
        grid=(Q // QB2,),
        in_specs=[pl.BlockSpec((QB2, NSEG), lambda q: (q, 0))],
        out_specs=pl.BlockSpec((QB2, K), lambda q: (q, 0)),
        out_shape=jax.ShapeDtypeStruct((Q, K), jnp.int32),
    )(m)

    return _phase3_sc(dist.reshape(Q * NPAD), seg_ids)
